# baseline (speedup 1.0000x reference)
import functools

import jax
import jax.numpy as jnp
from jax import lax
from jax.experimental import pallas as pl
from jax.experimental.pallas import tpu as pltpu
from jax.experimental.pallas import tpu_sc as plsc

E = 3_200_000
DIM = 16
NC, NS = 2, 16
NW = NC * NS
TILE = 128
NTILES = E // TILE
T = 5
CHUNK = T * TILE
NCHUNKS = NTILES // T
HALF = T * 1024
GROUPS = CHUNK // DIM

_mesh = plsc.VectorSubcoreMesh(core_axis_name="c", subcore_axis_name="s")


@functools.partial(
    pl.kernel,
    out_type=jax.ShapeDtypeStruct((E * DIM,), jnp.float32),
    mesh=_mesh,
    scratch_types=[
        pltpu.VMEM((2 * DIM,), jnp.float32),
        pltpu.VMEM((2, CHUNK), jnp.int32),
        pltpu.VMEM((2, 2 * HALF), jnp.float32),
        pltpu.SemaphoreType.DMA,
        pltpu.SemaphoreType.DMA,
    ],
    compiler_params=pltpu.CompilerParams(
        use_tc_tiling_on_sc=False, needs_layout_passes=False),
)
def _embed(mask_hbm, table_hbm, out_hbm, table_v, mask_v, rows_v, sem_in, sem_out):
    wid = lax.axis_index("s") * NC + lax.axis_index("c")
    nk = 156 + jnp.where(wid < 8, 1, 0).astype(jnp.int32)
    pltpu.sync_copy(table_hbm, table_v)
    t0 = table_v[pl.ds(0, DIM)]
    t1 = table_v[pl.ds(DIM, DIM)]
    s0 = [jnp.full((DIM,), t0[l], jnp.float32) for l in range(DIM)]
    s1 = [jnp.full((DIM,), t1[l], jnp.float32) for l in range(DIM)]

    def in_copy(k, slot):
        c = wid + NW * k
        return pltpu.async_copy(
            mask_hbm.at[pl.ds(c * CHUNK, CHUNK)], mask_v.at[slot], sem_in)

    def out_copy(k, slot):
        c = wid + NW * k
        for half in (0, 1):
            pltpu.async_copy(
                rows_v.at[slot, pl.ds(half * HALF, HALF)],
                out_hbm.at[pl.ds((half * NTILES + c * T) * 1024, HALF)],
                sem_out)

    def wait_in(slot):
        pltpu.make_async_copy(
            mask_hbm.at[pl.ds(0, CHUNK)], mask_v.at[slot], sem_in).wait()

    def wait_out(slot):
        for half in (0, 1):
            pltpu.make_async_copy(
                rows_v.at[slot, pl.ds(half * HALF, HALF)],
                out_hbm.at[pl.ds(half * HALF, HALF)], sem_out).wait()

    def compute(slot):
        rows = rows_v.at[slot]

        @plsc.parallel_loop(0, GROUPS, unroll=4)
        def _group(g):
            m = mask_v[slot, pl.ds(g * DIM, DIM)]
            p = m != 0
            base = (g // 8) * 1024 + (g % 8) * DIM
            for l in range(DIM):
                off = (l // 8) * HALF + (l % 8) * TILE + base
                rows[pl.ds(off, DIM)] = jnp.where(p, s1[l], s0[l])

    def step(k, slot):
        wait_in(slot)
        if isinstance(k, int):
            if k >= 2:
                wait_out(slot)
        else:
            @pl.when(k >= 2)
            def _():
                wait_out(slot)

        compute(slot)
        out_copy(k, slot)

        if not (isinstance(k, int) and k + 2 >= 158):
            @pl.when(k + 2 < nk)
            def _():
                in_copy(k + 2, slot)

    in_copy(0, 0)
    in_copy(1, 1)

    @pl.loop(0, 156, step=2)
    def _chunk(k):
        step(k, 0)
        step(k + 1, 1)

    @pl.when(wid < 8)
    def _():
        step(156, 0)

    wait_out(0)
    wait_out(1)


def kernel(bond_mask, table):
    flat = _embed(bond_mask, jnp.reshape(table, (2 * DIM,)))
    return (flat.reshape(2, NTILES, 8, TILE)
                .transpose(1, 3, 0, 2)
                .reshape(E, DIM))


# speedup vs baseline: 1.3194x; 1.3194x over previous
import functools

import jax
import jax.numpy as jnp
from jax import lax
from jax.experimental import pallas as pl
from jax.experimental.pallas import tpu as pltpu
from jax.experimental.pallas import tpu_sc as plsc

E = 3_200_000
DIM = 16
NC, NS = 2, 16
NW = NC * NS
TILE = 128
NTILES = E // TILE
T = 25
CHUNK = T * TILE
NCHUNKS = NTILES // T
HALF = T * 1024
GROUPS = CHUNK // DIM

_mesh = plsc.VectorSubcoreMesh(core_axis_name="c", subcore_axis_name="s")


@functools.partial(
    pl.kernel,
    out_type=jax.ShapeDtypeStruct((E * DIM,), jnp.float32),
    mesh=_mesh,
    scratch_types=[
        pltpu.VMEM((2 * DIM,), jnp.float32),
        pltpu.VMEM((2, CHUNK), jnp.int32),
        pltpu.VMEM((2, 2 * HALF), jnp.float32),
        pltpu.SemaphoreType.DMA,
        pltpu.SemaphoreType.DMA,
    ],
    compiler_params=pltpu.CompilerParams(
        use_tc_tiling_on_sc=False, needs_layout_passes=False),
)
def _embed(mask_hbm, table_hbm, out_hbm, table_v, mask_v, rows_v, sem_in, sem_out):
    wid = lax.axis_index("s") * NC + lax.axis_index("c")
    nk = 31 + jnp.where(wid < 8, 1, 0).astype(jnp.int32)
    pltpu.sync_copy(table_hbm, table_v)
    t0 = table_v[pl.ds(0, DIM)]
    t1 = table_v[pl.ds(DIM, DIM)]
    s0 = [jnp.full((DIM,), t0[l], jnp.float32) for l in range(DIM)]
    s1 = [jnp.full((DIM,), t1[l], jnp.float32) for l in range(DIM)]

    def in_copy(k, slot):
        c = wid + NW * k
        return pltpu.async_copy(
            mask_hbm.at[pl.ds(c * CHUNK, CHUNK)], mask_v.at[slot], sem_in)

    def out_copy(k, slot):
        c = wid + NW * k
        for half in (0, 1):
            pltpu.async_copy(
                rows_v.at[slot, pl.ds(half * HALF, HALF)],
                out_hbm.at[pl.ds((half * NTILES + c * T) * 1024, HALF)],
                sem_out)

    def wait_in(slot):
        pltpu.make_async_copy(
            mask_hbm.at[pl.ds(0, CHUNK)], mask_v.at[slot], sem_in).wait()

    def wait_out(slot):
        for half in (0, 1):
            pltpu.make_async_copy(
                rows_v.at[slot, pl.ds(half * HALF, HALF)],
                out_hbm.at[pl.ds(half * HALF, HALF)], sem_out).wait()

    def compute(slot):
        rows = rows_v.at[slot]

        @plsc.parallel_loop(0, GROUPS, unroll=4)
        def _group(g):
            m = mask_v[slot, pl.ds(g * DIM, DIM)]
            p = m != 0
            base = (g // 8) * 1024 + (g % 8) * DIM
            for l in range(DIM):
                off = (l // 8) * HALF + (l % 8) * TILE + base
                rows[pl.ds(off, DIM)] = jnp.where(p, s1[l], s0[l])

    def step(k, slot):
        wait_in(slot)
        if isinstance(k, int):
            if k >= 2:
                wait_out(slot)
        else:
            @pl.when(k >= 2)
            def _():
                wait_out(slot)

        compute(slot)
        out_copy(k, slot)

        if not (isinstance(k, int) and k + 2 >= 32):
            @pl.when(k + 2 < nk)
            def _():
                in_copy(k + 2, slot)

    in_copy(0, 0)
    in_copy(1, 1)

    @pl.loop(0, 30, step=2)
    def _chunk(k):
        step(k, 0)
        step(k + 1, 1)

    step(30, 0)

    @pl.when(wid < 8)
    def _():
        step(31, 1)

    wait_out(0)
    wait_out(1)


def kernel(bond_mask, table):
    flat = _embed(bond_mask, jnp.reshape(table, (2 * DIM,)))
    return (flat.reshape(2, NTILES, 8, TILE)
                .transpose(1, 3, 0, 2)
                .reshape(E, DIM))
